# Initial kernel scaffold; baseline (speedup 1.0000x reference)
import jax
import jax.numpy as jnp
from jax.experimental import pallas as pl


def kernel(q, k):
    raise NotImplementedError("write your pallas kernel here")


# speedup vs baseline: 1.4962x; 1.4962x over previous
import functools
import math

import jax
import jax.numpy as jnp
from jax.experimental import pallas as pl

SPARSE_TOP_K_ = 64


def _body(q_ref, k_ref, o_ref, *, bq, s, topk):
    qb = q_ref[0]
    kb = k_ref[0]
    scores = jax.lax.dot_general(
        qb, kb, (((1,), (1,)), ((), ())),
        preferred_element_type=jnp.float32,
    ) * (1.0 / math.sqrt(q_ref.shape[-1]))
    iota = jax.lax.broadcasted_iota(jnp.int32, (bq, s), 1)

    jcol = jax.lax.broadcasted_iota(jnp.int32, (bq, topk), 1)

    def step(i, carry):
        sc, out = carry
        m = jnp.max(sc, axis=1, keepdims=True)
        idx = jnp.min(jnp.where(sc == m, iota, s), axis=1, keepdims=True)
        sc = jnp.where(iota == idx, -jnp.inf, sc)
        out = jnp.where(jcol == i, idx, out)
        return sc, out

    out0 = jnp.zeros((bq, topk), jnp.int32)
    _, out = jax.lax.fori_loop(0, topk, step, (scores, out0))
    o_ref[0] = out


def kernel(q, k):
    B, H, S, D = q.shape
    topk = min(SPARSE_TOP_K_, S)
    bq = 256
    qf = q.reshape(B * H, S, D)
    kf = k.reshape(B * H, S, D)
    out = pl.pallas_call(
        functools.partial(_body, bq=bq, s=S, topk=topk),
        grid=(B * H, S // bq),
        in_specs=[
            pl.BlockSpec((1, bq, D), lambda i, j: (i, j, 0)),
            pl.BlockSpec((1, S, D), lambda i, j: (i, 0, 0)),
        ],
        out_specs=pl.BlockSpec((1, bq, topk), lambda i, j: (i, j, 0)),
        out_shape=jax.ShapeDtypeStruct((B * H, S, topk), jnp.int32),
    )(qf, kf)
    return out.reshape(B, H, S, topk).astype(jnp.int64)


# speedup vs baseline: 2.8430x; 1.9002x over previous
import functools
import math

import jax
import jax.numpy as jnp
import numpy as np
from jax.experimental import pallas as pl
from jax.sharding import Mesh, PartitionSpec as P

try:
    from jax.experimental.shard_map import shard_map as _shard_map
except ImportError:
    _shard_map = jax.shard_map

SPARSE_TOP_K_ = 64


def _body(q_ref, k_ref, o_ref, *, bq, s, topk):
    qb = q_ref[0]
    kb = k_ref[0]
    scores = jax.lax.dot_general(
        qb, kb, (((1,), (1,)), ((), ())),
        preferred_element_type=jnp.float32,
    ) * (1.0 / math.sqrt(q_ref.shape[-1]))
    iota = jax.lax.broadcasted_iota(jnp.int32, (bq, s), 1)

    jcol = jax.lax.broadcasted_iota(jnp.int32, (bq, topk), 1)

    def step(i, carry):
        sc, out = carry
        m = jnp.max(sc, axis=1, keepdims=True)
        idx = jnp.min(jnp.where(sc == m, iota, s), axis=1, keepdims=True)
        sc = jnp.where(iota == idx, -jnp.inf, sc)
        out = jnp.where(jcol == i, idx, out)
        return sc, out

    out0 = jnp.zeros((bq, topk), jnp.int32)
    _, out = jax.lax.fori_loop(0, topk, step, (scores, out0))
    o_ref[0] = out


def _topk_local(qf, kf, *, bq, topk):
    BH, S, D = qf.shape
    return pl.pallas_call(
        functools.partial(_body, bq=bq, s=S, topk=topk),
        grid=(BH, S // bq),
        in_specs=[
            pl.BlockSpec((1, bq, D), lambda i, j: (i, j, 0)),
            pl.BlockSpec((1, S, D), lambda i, j: (i, 0, 0)),
        ],
        out_specs=pl.BlockSpec((1, bq, topk), lambda i, j: (i, j, 0)),
        out_shape=jax.ShapeDtypeStruct((BH, S, topk), jnp.int32),
    )(qf, kf)


def kernel(q, k):
    B, H, S, D = q.shape
    topk = min(SPARSE_TOP_K_, S)
    bq = 256
    qf = q.reshape(B * H, S, D)
    kf = k.reshape(B * H, S, D)
    fn = functools.partial(_topk_local, bq=bq, topk=topk)
    devs = jax.devices()
    if len(devs) >= 2 and (B * H) % 2 == 0:
        mesh = Mesh(np.asarray(devs[:2]), ("x",))
        out = _shard_map(
            fn, mesh=mesh,
            in_specs=(P("x"), P("x")),
            out_specs=P("x"),
            check_rep=False,
        )(qf, kf)
    else:
        out = fn(qf, kf)
    return out.reshape(B, H, S, topk).astype(jnp.int64)
